# baseline (speedup 1.0000x reference)
import functools
import math

import jax
import jax.numpy as jnp
from jax import lax
from jax.experimental import pallas as pl
from jax.experimental.pallas import tpu as pltpu
from jax.experimental.pallas import tpu_sc as plsc

_N = 2_000_000
_G = 1000
_LW = 1.0
_SIZE = _LW / _G
_DZ = _SIZE / 2.0
_A = 0.0136

_INFO = plsc.get_sparse_core_info()
_NC = _INFO.num_cores
_NS = _INFO.num_subcores
_NW = _NC * _NS
_L = 16

_TB = 131072
_SLICE_A = 8 * _TB
_SLICE_B = _N - _SLICE_A

_SENT = _G * _G

_mesh = plsc.VectorSubcoreMesh(core_axis_name="c", subcore_axis_name="s")


def _quantize(xv, yv):
  ix = jnp.minimum(jnp.maximum((xv / _SIZE).astype(jnp.int32), 0), _G - 1)
  iy = jnp.minimum(jnp.maximum((yv / _SIZE).astype(jnp.int32), 0), _G - 1)
  return ix * _G + iy


def _make_sc_rl(C, OFF, NCH, TAIL):
  K = (NCH + _NW - 1) // _NW
  scratch = (
      [pltpu.VMEM((C,), jnp.float32) for _ in range(2)]
      + [pltpu.VMEM((C,), jnp.float32) for _ in range(2)]
      + [pltpu.VMEM((C,), jnp.int32) for _ in range(2)]
      + [pltpu.VMEM((C,), jnp.float32) for _ in range(2)]
  )
  if TAIL:
    scratch += [pltpu.VMEM((TAIL,), jnp.float32),
                pltpu.VMEM((TAIL,), jnp.float32),
                pltpu.VMEM((TAIL,), jnp.int32),
                pltpu.VMEM((TAIL,), jnp.float32)]
  scratch += [pltpu.SemaphoreType.DMA for _ in range(5)]

  @functools.partial(
      pl.kernel, mesh=_mesh,
      out_type=jax.ShapeDtypeStruct((_N,), jnp.float32),
      scratch_types=scratch,
  )
  def k(x_hbm, y_hbm, tab_hbm, out_hbm, *refs):
    xb = refs[0:2]
    yb = refs[2:4]
    ib = refs[4:6]
    gb = refs[6:8]
    pos = 8
    if TAIL:
      xbt, ybt, ibt, gbt = refs[8:12]
      pos = 12
    semL = refs[pos:pos + 2]
    semG = refs[pos + 2]
    semS = refs[pos + 3:pos + 5]

    wid = lax.axis_index("s") * _NC + lax.axis_index("c")

    def cbase(kk):
      return OFF + (wid + kk * _NW) * C

    def guard(kk):
      return wid + kk * _NW < NCH

    @pl.when(guard(0))
    def _():
      pltpu.async_copy(x_hbm.at[pl.ds(cbase(0), C)], xb[0], semL[0])
      pltpu.async_copy(y_hbm.at[pl.ds(cbase(0), C)], yb[0], semL[0])

    for kk in range(K):
      b = kk % 2
      nb = (kk + 1) % 2

      @pl.when(guard(kk))
      def _(kk=kk, b=b):
        pltpu.make_async_copy(x_hbm.at[pl.ds(cbase(kk), C)], xb[b],
                              semL[b]).wait()
        pltpu.make_async_copy(y_hbm.at[pl.ds(cbase(kk), C)], yb[b],
                              semL[b]).wait()

      if kk + 1 < K:
        @pl.when(guard(kk + 1))
        def _(kk=kk, nb=nb):
          pltpu.async_copy(x_hbm.at[pl.ds(cbase(kk + 1), C)], xb[nb],
                           semL[nb])
          pltpu.async_copy(y_hbm.at[pl.ds(cbase(kk + 1), C)], yb[nb],
                           semL[nb])

      @pl.when(guard(kk))
      def _(kk=kk, b=b):
        def step(j, c2):
          sl = pl.ds(j * _L, _L)
          ib[b][sl] = _quantize(xb[b][sl], yb[b][sl])
          return c2

        lax.fori_loop(0, C // _L, step, 0)

      if kk >= 2:
        @pl.when(guard(kk - 2))
        def _(kk=kk, b=b):
          pltpu.make_async_copy(gb[b], out_hbm.at[pl.ds(cbase(kk - 2), C)],
                                semS[b]).wait()

      @pl.when(guard(kk))
      def _(kk=kk, b=b):
        pltpu.async_copy(tab_hbm.at[ib[b]], gb[b], semG).wait()
        pltpu.async_copy(gb[b], out_hbm.at[pl.ds(cbase(kk), C)], semS[b])

    for kk in (K - 2, K - 1):
      b = kk % 2

      @pl.when(guard(kk))
      def _(kk=kk, b=b):
        pltpu.make_async_copy(gb[b], out_hbm.at[pl.ds(cbase(kk), C)],
                              semS[b]).wait()

    if TAIL:
      @pl.when(wid == NCH % _NW)
      def _():
        tbase = OFF + NCH * C
        pltpu.sync_copy(x_hbm.at[pl.ds(tbase, TAIL)], xbt)
        pltpu.sync_copy(y_hbm.at[pl.ds(tbase, TAIL)], ybt)

        def step(j, c2):
          sl = pl.ds(j * _L, _L)
          ibt[sl] = _quantize(xbt[sl], ybt[sl])
          return c2

        lax.fori_loop(0, TAIL // _L, step, 0)
        pltpu.async_copy(tab_hbm.at[ibt], gbt, semG).wait()
        pltpu.sync_copy(gbt, out_hbm.at[pl.ds(tbase, TAIL)])

  return k


def _make_sc_hits(C, OFF, NCH, TAIL):
  K = (NCH + _NW - 1) // _NW
  scratch = (
      [pltpu.VMEM((C,), jnp.float32) for _ in range(2)]
      + [pltpu.VMEM((C,), jnp.float32) for _ in range(2)]
      + [pltpu.VMEM((C,), jnp.float32) for _ in range(2)]
      + [pltpu.VMEM((C,), jnp.float32) for _ in range(2)]
      + [pltpu.VMEM((C,), jnp.int32) for _ in range(2)]
      + [pltpu.VMEM((C,), jnp.float32) for _ in range(2)]
      + [pltpu.VMEM((C,), jnp.float32) for _ in range(2)]
      + [pltpu.VMEM((C,), jnp.float32) for _ in range(2)]
  )
  if TAIL:
    scratch += ([pltpu.VMEM((TAIL,), jnp.float32) for _ in range(4)]
                + [pltpu.VMEM((TAIL,), jnp.int32)]
                + [pltpu.VMEM((TAIL,), jnp.float32) for _ in range(3)])
  scratch += [pltpu.SemaphoreType.DMA for _ in range(5)]

  @functools.partial(
      pl.kernel, mesh=_mesh,
      out_type=(jax.ShapeDtypeStruct((_N,), jnp.float32),
                jax.ShapeDtypeStruct((_N,), jnp.float32)),
      scratch_types=scratch,
  )
  def k(xp_hbm, yp_hbm, nx_hbm, ny_hbm, tab_hbm, hx_hbm, hy_hbm, *refs):
    xpb = refs[0:2]
    ypb = refs[2:4]
    nxb = refs[4:6]
    nyb = refs[6:8]
    f2b = refs[8:10]
    resb = refs[10:12]
    hxb = refs[12:14]
    hyb = refs[14:16]
    pos = 16
    if TAIL:
      xpt, ypt, nxt, nyt, f2t, rest, hxt, hyt = refs[16:24]
      pos = 24
    semL = refs[pos:pos + 2]
    semG = refs[pos + 2]
    semS = refs[pos + 3:pos + 5]

    wid = lax.axis_index("s") * _NC + lax.axis_index("c")
    ins = (xp_hbm, yp_hbm, nx_hbm, ny_hbm)

    def bufs(b):
      return (xpb[b], ypb[b], nxb[b], nyb[b])

    def cbase(kk):
      return OFF + (wid + kk * _NW) * C

    def guard(kk):
      return wid + kk * _NW < NCH

    def issue_loads(kk, b):
      sl = pl.ds(cbase(kk), C)
      for src, dst in zip(ins, bufs(b)):
        pltpu.async_copy(src.at[sl], dst, semL[b])

    def wait_loads(kk, b):
      sl = pl.ds(cbase(kk), C)
      for src, dst in zip(ins, bufs(b)):
        pltpu.make_async_copy(src.at[sl], dst, semL[b]).wait()

    @pl.when(guard(0))
    def _():
      issue_loads(0, 0)

    for kk in range(K):
      b = kk % 2
      nb = (kk + 1) % 2

      @pl.when(guard(kk))
      def _(kk=kk, b=b):
        wait_loads(kk, b)

      if kk + 1 < K:
        @pl.when(guard(kk + 1))
        def _(kk=kk, nb=nb):
          issue_loads(kk + 1, nb)

      if kk >= 2:
        @pl.when(guard(kk - 2))
        def _(kk=kk, b=b):
          pltpu.make_async_copy(hxb[b], hx_hbm.at[pl.ds(cbase(kk - 2), C)],
                                semS[b]).wait()
          pltpu.make_async_copy(hyb[b], hy_hbm.at[pl.ds(cbase(kk - 2), C)],
                                semS[b]).wait()

      @pl.when(guard(kk))
      def _(kk=kk, b=b):
        def qstep(j, c2):
          sl = pl.ds(j * _L, _L)
          xv = xpb[b][sl]
          yv = ypb[b][sl]
          inb = (xv >= 0.0) & (xv < _LW) & (yv >= 0.0) & (yv < _LW)
          f2b[b][sl] = jnp.where(inb, _quantize(xv, yv), _SENT)
          return c2

        lax.fori_loop(0, C // _L, qstep, 0)
        pltpu.async_copy(tab_hbm.at[f2b[b]], resb[b], semG).wait()

        def step(j, c2):
          sl = pl.ds(j * _L, _L)
          d = jnp.abs(resb[b][sl]) + 1e-17
          hxb[b][sl] = xpb[b][sl] + nxb[b][sl] / d
          hyb[b][sl] = ypb[b][sl] + nyb[b][sl] / d
          return c2

        lax.fori_loop(0, C // _L, step, 0)
        pltpu.async_copy(hxb[b], hx_hbm.at[pl.ds(cbase(kk), C)], semS[b])
        pltpu.async_copy(hyb[b], hy_hbm.at[pl.ds(cbase(kk), C)], semS[b])

    for kk in (K - 2, K - 1):
      b = kk % 2

      @pl.when(guard(kk))
      def _(kk=kk, b=b):
        pltpu.make_async_copy(hxb[b], hx_hbm.at[pl.ds(cbase(kk), C)],
                              semS[b]).wait()
        pltpu.make_async_copy(hyb[b], hy_hbm.at[pl.ds(cbase(kk), C)],
                              semS[b]).wait()

    if TAIL:
      @pl.when(wid == NCH % _NW)
      def _():
        tbase = OFF + NCH * C
        sl = pl.ds(tbase, TAIL)
        pltpu.sync_copy(xp_hbm.at[sl], xpt)
        pltpu.sync_copy(yp_hbm.at[sl], ypt)
        pltpu.sync_copy(nx_hbm.at[sl], nxt)
        pltpu.sync_copy(ny_hbm.at[sl], nyt)

        def qstep(j, c2):
          s = pl.ds(j * _L, _L)
          xv = xpt[s]
          yv = ypt[s]
          inb = (xv >= 0.0) & (xv < _LW) & (yv >= 0.0) & (yv < _LW)
          f2t[s] = jnp.where(inb, _quantize(xv, yv), _SENT)
          return c2

        lax.fori_loop(0, TAIL // _L, qstep, 0)
        pltpu.async_copy(tab_hbm.at[f2t], rest, semG).wait()

        def step(j, c2):
          s = pl.ds(j * _L, _L)
          d = jnp.abs(rest[s]) + 1e-17
          hxt[s] = xpt[s] + nxt[s] / d
          hyt[s] = ypt[s] + nyt[s] / d
          return c2

        lax.fori_loop(0, TAIL // _L, step, 0)
        pltpu.sync_copy(hxt, hx_hbm.at[sl])
        pltpu.sync_copy(hyt, hy_hbm.at[sl])

  return k


def _tc_math_body(x_ref, y_ref, th_ref, tx_ref, ty_ref, p_ref, z1_ref, z2_ref,
                  u_ref, rl_ref, xp_ref, yp_ref):
  x = x_ref[...]
  y = y_ref[...]
  theta = th_ref[...]
  theta_x = tx_ref[...]
  theta_y = ty_ref[...]
  p = p_ref[...]
  z1 = z1_ref[...]
  z2 = z2_ref[...]
  u = u_ref[...]
  rl = rl_ref[...]

  mask = (x >= 0.0) & (x < _LW) & (y >= 0.0) & (y < _LW)
  x0 = _DZ / (rl * jnp.cos(theta))
  theta0 = _A / p * jnp.sqrt(x0)
  phi = u * 2.0 * math.pi
  dh = _DZ * jnp.sin(theta0) * (z1 / math.sqrt(12.0) + z2 / 2.0)
  dx = math.sqrt(2.0) * dh * jnp.cos(phi) * jnp.cos(theta_x)
  dy = math.sqrt(2.0) * dh * jnp.sin(phi) * jnp.cos(theta_y)
  xn = jnp.where(mask, x + dx, x)
  yn = jnp.where(mask, y + dy, y)
  xn = xn + _DZ * jnp.tan(theta_x)
  yn = yn + _DZ * jnp.tan(theta_y)

  xp_ref[...] = xn
  yp_ref[...] = yn


def _make_tc_math(grid, block_off):
  spec = pl.BlockSpec((_TB,), lambda i, o=block_off: (i + o,))
  return pl.pallas_call(
      _tc_math_body,
      grid=(grid,),
      in_specs=[spec] * 10,
      out_specs=[spec] * 2,
      out_shape=[
          jax.ShapeDtypeStruct((_N,), jnp.float32),
          jax.ShapeDtypeStruct((_N,), jnp.float32),
      ],
  )


_S = 524288
_SLICES = (
    (0, (8192, 64, 0), (4096, 128, 0), (4, 0)),
    (_S, (8192, 64, 0), (4096, 128, 0), (4, 4)),
    (2 * _S, (8192, 64, 0), (4096, 128, 0), (4, 8)),
    (3 * _S, (8192, 52, 1152), (4096, 104, 1152), (4, 12)),
)

_sc_rl_k = [_make_sc_rl(c, off, nch, tl)
            for off, (c, nch, tl), _, _ in _SLICES]
_sc_hits_k = [_make_sc_hits(c, off, nch, tl)
              for off, _, (c, nch, tl), _ in _SLICES]
_tc_math_k = [_make_tc_math(g, bo) for _, _, _, (g, bo) in _SLICES]


def kernel(x, y, theta, theta_x, theta_y, p, z1a, z2a, ua, z1b, z2b, ub,
           nx, ny, resolution, efficiency, rad_length):
  tab1 = rad_length.reshape(-1)
  tab2 = jnp.concatenate(
      [resolution.reshape(-1), jnp.zeros((8,), jnp.float32)])

  rls = [k(x, y, tab1) for k in _sc_rl_k]
  xys = [k(x, y, theta, theta_x, theta_y, p, z1a, z2a, ua, rl)
         for k, rl in zip(_tc_math_k, rls)]
  hs = [k(xp, yp, nx, ny, tab2)
        for k, (xp, yp) in zip(_sc_hits_k, xys)]

  bounds = [off for off, _, _, _ in _SLICES] + [_N]
  hx = jnp.concatenate([h[0][bounds[i]:bounds[i + 1]]
                        for i, h in enumerate(hs)])
  hy = jnp.concatenate([h[1][bounds[i]:bounds[i + 1]]
                        for i, h in enumerate(hs)])
  return jnp.stack([hx, hy], axis=1)


# speedup vs baseline: 1.0177x; 1.0177x over previous
import functools
import math

import jax
import jax.numpy as jnp
from jax import lax
from jax.experimental import pallas as pl
from jax.experimental.pallas import tpu as pltpu
from jax.experimental.pallas import tpu_sc as plsc

_N = 2_000_000
_G = 1000
_LW = 1.0
_SIZE = _LW / _G
_DZ = _SIZE / 2.0
_A = 0.0136

_INFO = plsc.get_sparse_core_info()
_NC = _INFO.num_cores
_NS = _INFO.num_subcores
_NW = _NC * _NS
_L = 16

_TB = 131072
_SLICE_A = 8 * _TB
_SLICE_B = _N - _SLICE_A

_SENT = _G * _G

_mesh = plsc.VectorSubcoreMesh(core_axis_name="c", subcore_axis_name="s")


def _quantize(xv, yv):
  ix = jnp.minimum(jnp.maximum((xv / _SIZE).astype(jnp.int32), 0), _G - 1)
  iy = jnp.minimum(jnp.maximum((yv / _SIZE).astype(jnp.int32), 0), _G - 1)
  return ix * _G + iy


def _make_sc_rl(C, OFF, NCH, TAIL):
  K = (NCH + _NW - 1) // _NW
  scratch = (
      [pltpu.VMEM((C,), jnp.float32) for _ in range(2)]
      + [pltpu.VMEM((C,), jnp.float32) for _ in range(2)]
      + [pltpu.VMEM((C,), jnp.int32) for _ in range(2)]
      + [pltpu.VMEM((C,), jnp.float32) for _ in range(2)]
  )
  if TAIL:
    scratch += [pltpu.VMEM((TAIL,), jnp.float32),
                pltpu.VMEM((TAIL,), jnp.float32),
                pltpu.VMEM((TAIL,), jnp.int32),
                pltpu.VMEM((TAIL,), jnp.float32)]
  scratch += [pltpu.SemaphoreType.DMA for _ in range(5)]

  @functools.partial(
      pl.kernel, mesh=_mesh,
      out_type=jax.ShapeDtypeStruct((_N,), jnp.float32),
      scratch_types=scratch,
  )
  def k(x_hbm, y_hbm, tab_hbm, out_hbm, *refs):
    xb = refs[0:2]
    yb = refs[2:4]
    ib = refs[4:6]
    gb = refs[6:8]
    pos = 8
    if TAIL:
      xbt, ybt, ibt, gbt = refs[8:12]
      pos = 12
    semL = refs[pos:pos + 2]
    semG = refs[pos + 2]
    semS = refs[pos + 3:pos + 5]

    wid = lax.axis_index("s") * _NC + lax.axis_index("c")

    def cbase(kk):
      return OFF + (wid + kk * _NW) * C

    def guard(kk):
      return wid + kk * _NW < NCH

    @pl.when(guard(0))
    def _():
      pltpu.async_copy(x_hbm.at[pl.ds(cbase(0), C)], xb[0], semL[0])
      pltpu.async_copy(y_hbm.at[pl.ds(cbase(0), C)], yb[0], semL[0])

    for kk in range(K):
      b = kk % 2
      nb = (kk + 1) % 2

      @pl.when(guard(kk))
      def _(kk=kk, b=b):
        pltpu.make_async_copy(x_hbm.at[pl.ds(cbase(kk), C)], xb[b],
                              semL[b]).wait()
        pltpu.make_async_copy(y_hbm.at[pl.ds(cbase(kk), C)], yb[b],
                              semL[b]).wait()

      if kk + 1 < K:
        @pl.when(guard(kk + 1))
        def _(kk=kk, nb=nb):
          pltpu.async_copy(x_hbm.at[pl.ds(cbase(kk + 1), C)], xb[nb],
                           semL[nb])
          pltpu.async_copy(y_hbm.at[pl.ds(cbase(kk + 1), C)], yb[nb],
                           semL[nb])

      @pl.when(guard(kk))
      def _(kk=kk, b=b):
        def step(j, c2):
          sl = pl.ds(j * _L, _L)
          ib[b][sl] = _quantize(xb[b][sl], yb[b][sl])
          return c2

        lax.fori_loop(0, C // _L, step, 0)

      if kk >= 2:
        @pl.when(guard(kk - 2))
        def _(kk=kk, b=b):
          pltpu.make_async_copy(gb[b], out_hbm.at[pl.ds(cbase(kk - 2), C)],
                                semS[b]).wait()

      @pl.when(guard(kk))
      def _(kk=kk, b=b):
        pltpu.async_copy(tab_hbm.at[ib[b]], gb[b], semG).wait()
        pltpu.async_copy(gb[b], out_hbm.at[pl.ds(cbase(kk), C)], semS[b])

    for kk in (K - 2, K - 1):
      b = kk % 2

      @pl.when(guard(kk))
      def _(kk=kk, b=b):
        pltpu.make_async_copy(gb[b], out_hbm.at[pl.ds(cbase(kk), C)],
                              semS[b]).wait()

    if TAIL:
      @pl.when(wid == NCH % _NW)
      def _():
        tbase = OFF + NCH * C
        pltpu.sync_copy(x_hbm.at[pl.ds(tbase, TAIL)], xbt)
        pltpu.sync_copy(y_hbm.at[pl.ds(tbase, TAIL)], ybt)

        def step(j, c2):
          sl = pl.ds(j * _L, _L)
          ibt[sl] = _quantize(xbt[sl], ybt[sl])
          return c2

        lax.fori_loop(0, TAIL // _L, step, 0)
        pltpu.async_copy(tab_hbm.at[ibt], gbt, semG).wait()
        pltpu.sync_copy(gbt, out_hbm.at[pl.ds(tbase, TAIL)])

  return k


def _make_sc_hits(C, OFF, NCH, TAIL):
  K = (NCH + _NW - 1) // _NW
  scratch = (
      [pltpu.VMEM((C,), jnp.float32) for _ in range(2)]
      + [pltpu.VMEM((C,), jnp.float32) for _ in range(2)]
      + [pltpu.VMEM((C,), jnp.float32) for _ in range(2)]
      + [pltpu.VMEM((C,), jnp.float32) for _ in range(2)]
      + [pltpu.VMEM((C,), jnp.int32) for _ in range(2)]
      + [pltpu.VMEM((C,), jnp.float32) for _ in range(2)]
      + [pltpu.VMEM((C,), jnp.float32) for _ in range(2)]
      + [pltpu.VMEM((C,), jnp.float32) for _ in range(2)]
  )
  if TAIL:
    scratch += ([pltpu.VMEM((TAIL,), jnp.float32) for _ in range(4)]
                + [pltpu.VMEM((TAIL,), jnp.int32)]
                + [pltpu.VMEM((TAIL,), jnp.float32) for _ in range(3)])
  scratch += [pltpu.SemaphoreType.DMA for _ in range(5)]

  @functools.partial(
      pl.kernel, mesh=_mesh,
      out_type=(jax.ShapeDtypeStruct((_N,), jnp.float32),
                jax.ShapeDtypeStruct((_N,), jnp.float32)),
      scratch_types=scratch,
  )
  def k(xp_hbm, yp_hbm, nx_hbm, ny_hbm, tab_hbm, hx_hbm, hy_hbm, *refs):
    xpb = refs[0:2]
    ypb = refs[2:4]
    nxb = refs[4:6]
    nyb = refs[6:8]
    f2b = refs[8:10]
    resb = refs[10:12]
    hxb = refs[12:14]
    hyb = refs[14:16]
    pos = 16
    if TAIL:
      xpt, ypt, nxt, nyt, f2t, rest, hxt, hyt = refs[16:24]
      pos = 24
    semL = refs[pos:pos + 2]
    semG = refs[pos + 2]
    semS = refs[pos + 3:pos + 5]

    wid = lax.axis_index("s") * _NC + lax.axis_index("c")
    ins = (xp_hbm, yp_hbm, nx_hbm, ny_hbm)

    def bufs(b):
      return (xpb[b], ypb[b], nxb[b], nyb[b])

    def cbase(kk):
      return OFF + (wid + kk * _NW) * C

    def guard(kk):
      return wid + kk * _NW < NCH

    def issue_loads(kk, b):
      sl = pl.ds(cbase(kk), C)
      for src, dst in zip(ins, bufs(b)):
        pltpu.async_copy(src.at[sl], dst, semL[b])

    def wait_loads(kk, b):
      sl = pl.ds(cbase(kk), C)
      for src, dst in zip(ins, bufs(b)):
        pltpu.make_async_copy(src.at[sl], dst, semL[b]).wait()

    @pl.when(guard(0))
    def _():
      issue_loads(0, 0)

    for kk in range(K):
      b = kk % 2
      nb = (kk + 1) % 2

      @pl.when(guard(kk))
      def _(kk=kk, b=b):
        wait_loads(kk, b)

      if kk + 1 < K:
        @pl.when(guard(kk + 1))
        def _(kk=kk, nb=nb):
          issue_loads(kk + 1, nb)

      if kk >= 2:
        @pl.when(guard(kk - 2))
        def _(kk=kk, b=b):
          pltpu.make_async_copy(hxb[b], hx_hbm.at[pl.ds(cbase(kk - 2), C)],
                                semS[b]).wait()
          pltpu.make_async_copy(hyb[b], hy_hbm.at[pl.ds(cbase(kk - 2), C)],
                                semS[b]).wait()

      @pl.when(guard(kk))
      def _(kk=kk, b=b):
        def qstep(j, c2):
          sl = pl.ds(j * _L, _L)
          xv = xpb[b][sl]
          yv = ypb[b][sl]
          inb = (xv >= 0.0) & (xv < _LW) & (yv >= 0.0) & (yv < _LW)
          f2b[b][sl] = jnp.where(inb, _quantize(xv, yv), _SENT)
          return c2

        lax.fori_loop(0, C // _L, qstep, 0)
        pltpu.async_copy(tab_hbm.at[f2b[b]], resb[b], semG).wait()

        def step(j, c2):
          sl = pl.ds(j * _L, _L)
          d = jnp.abs(resb[b][sl]) + 1e-17
          hxb[b][sl] = xpb[b][sl] + nxb[b][sl] / d
          hyb[b][sl] = ypb[b][sl] + nyb[b][sl] / d
          return c2

        lax.fori_loop(0, C // _L, step, 0)
        pltpu.async_copy(hxb[b], hx_hbm.at[pl.ds(cbase(kk), C)], semS[b])
        pltpu.async_copy(hyb[b], hy_hbm.at[pl.ds(cbase(kk), C)], semS[b])

    for kk in (K - 2, K - 1):
      b = kk % 2

      @pl.when(guard(kk))
      def _(kk=kk, b=b):
        pltpu.make_async_copy(hxb[b], hx_hbm.at[pl.ds(cbase(kk), C)],
                              semS[b]).wait()
        pltpu.make_async_copy(hyb[b], hy_hbm.at[pl.ds(cbase(kk), C)],
                              semS[b]).wait()

    if TAIL:
      @pl.when(wid == NCH % _NW)
      def _():
        tbase = OFF + NCH * C
        sl = pl.ds(tbase, TAIL)
        pltpu.sync_copy(xp_hbm.at[sl], xpt)
        pltpu.sync_copy(yp_hbm.at[sl], ypt)
        pltpu.sync_copy(nx_hbm.at[sl], nxt)
        pltpu.sync_copy(ny_hbm.at[sl], nyt)

        def qstep(j, c2):
          s = pl.ds(j * _L, _L)
          xv = xpt[s]
          yv = ypt[s]
          inb = (xv >= 0.0) & (xv < _LW) & (yv >= 0.0) & (yv < _LW)
          f2t[s] = jnp.where(inb, _quantize(xv, yv), _SENT)
          return c2

        lax.fori_loop(0, TAIL // _L, qstep, 0)
        pltpu.async_copy(tab_hbm.at[f2t], rest, semG).wait()

        def step(j, c2):
          s = pl.ds(j * _L, _L)
          d = jnp.abs(rest[s]) + 1e-17
          hxt[s] = xpt[s] + nxt[s] / d
          hyt[s] = ypt[s] + nyt[s] / d
          return c2

        lax.fori_loop(0, TAIL // _L, step, 0)
        pltpu.sync_copy(hxt, hx_hbm.at[sl])
        pltpu.sync_copy(hyt, hy_hbm.at[sl])

  return k


def _tc_math_body(x_ref, y_ref, th_ref, tx_ref, ty_ref, p_ref, z1_ref, z2_ref,
                  u_ref, rl_ref, xp_ref, yp_ref):
  x = x_ref[...]
  y = y_ref[...]
  theta = th_ref[...]
  theta_x = tx_ref[...]
  theta_y = ty_ref[...]
  p = p_ref[...]
  z1 = z1_ref[...]
  z2 = z2_ref[...]
  u = u_ref[...]
  rl = rl_ref[...]

  mask = (x >= 0.0) & (x < _LW) & (y >= 0.0) & (y < _LW)
  x0 = _DZ / (rl * jnp.cos(theta))
  theta0 = _A / p * jnp.sqrt(x0)
  phi = u * 2.0 * math.pi
  dh = _DZ * jnp.sin(theta0) * (z1 / math.sqrt(12.0) + z2 / 2.0)
  dx = math.sqrt(2.0) * dh * jnp.cos(phi) * jnp.cos(theta_x)
  dy = math.sqrt(2.0) * dh * jnp.sin(phi) * jnp.cos(theta_y)
  xn = jnp.where(mask, x + dx, x)
  yn = jnp.where(mask, y + dy, y)
  xn = xn + _DZ * jnp.tan(theta_x)
  yn = yn + _DZ * jnp.tan(theta_y)

  xp_ref[...] = xn
  yp_ref[...] = yn


def _make_tc_math(grid, block_off):
  spec = pl.BlockSpec((_TB,), lambda i, o=block_off: (i + o,))
  return pl.pallas_call(
      _tc_math_body,
      grid=(grid,),
      in_specs=[spec] * 10,
      out_specs=[spec] * 2,
      out_shape=[
          jax.ShapeDtypeStruct((_N,), jnp.float32),
          jax.ShapeDtypeStruct((_N,), jnp.float32),
      ],
  )


_SLICES = (
    (0, (8192, 128, 0), (4096, 256, 0), (8, 0)),
    (_SLICE_A, (8192, 116, 1152), (4096, 232, 1152), (8, 8)),
)

_sc_rl_k = [_make_sc_rl(c, off, nch, tl)
            for off, (c, nch, tl), _, _ in _SLICES]
_sc_hits_k = [_make_sc_hits(c, off, nch, tl)
              for off, _, (c, nch, tl), _ in _SLICES]
_tc_math_k = [_make_tc_math(g, bo) for _, _, _, (g, bo) in _SLICES]


def kernel(x, y, theta, theta_x, theta_y, p, z1a, z2a, ua, z1b, z2b, ub,
           nx, ny, resolution, efficiency, rad_length):
  tab1 = rad_length.reshape(-1)
  tab2 = jnp.concatenate(
      [resolution.reshape(-1), jnp.zeros((8,), jnp.float32)])

  rls = [k(x, y, tab1) for k in _sc_rl_k]
  xys = [k(x, y, theta, theta_x, theta_y, p, z1a, z2a, ua, rl)
         for k, rl in zip(_tc_math_k, rls)]
  hs = [k(xp, yp, nx, ny, tab2)
        for k, (xp, yp) in zip(_sc_hits_k, xys)]

  bounds = [off for off, _, _, _ in _SLICES] + [_N]
  hx = jnp.concatenate([h[0][bounds[i]:bounds[i + 1]]
                        for i, h in enumerate(hs)])
  hy = jnp.concatenate([h[1][bounds[i]:bounds[i + 1]]
                        for i, h in enumerate(hs)])
  return jnp.stack([hx, hy], axis=1)


# speedup vs baseline: 1.0447x; 1.0266x over previous
import functools
import math

import jax
import jax.numpy as jnp
from jax import lax
from jax.experimental import pallas as pl
from jax.experimental.pallas import tpu as pltpu
from jax.experimental.pallas import tpu_sc as plsc

_N = 2_000_000
_G = 1000
_LW = 1.0
_SIZE = _LW / _G
_DZ = _SIZE / 2.0
_A = 0.0136

_INFO = plsc.get_sparse_core_info()
_NC = _INFO.num_cores
_NS = _INFO.num_subcores
_NW = _NC * _NS
_L = 16

_TB = 131072
_SLICE_A = 8 * _TB
_SLICE_B = _N - _SLICE_A

_SENT = _G * _G

_mesh = plsc.VectorSubcoreMesh(core_axis_name="c", subcore_axis_name="s")


def _quantize(xv, yv):
  ix = jnp.minimum(jnp.maximum((xv / _SIZE).astype(jnp.int32), 0), _G - 1)
  iy = jnp.minimum(jnp.maximum((yv / _SIZE).astype(jnp.int32), 0), _G - 1)
  return ix * _G + iy


def _make_sc_rl(C, OFF, NCH, TAIL):
  K = (NCH + _NW - 1) // _NW
  scratch = (
      [pltpu.VMEM((C,), jnp.float32) for _ in range(2)]
      + [pltpu.VMEM((C,), jnp.float32) for _ in range(2)]
      + [pltpu.VMEM((C,), jnp.int32) for _ in range(2)]
      + [pltpu.VMEM((C,), jnp.float32) for _ in range(2)]
  )
  if TAIL:
    scratch += [pltpu.VMEM((TAIL,), jnp.float32),
                pltpu.VMEM((TAIL,), jnp.float32),
                pltpu.VMEM((TAIL,), jnp.int32),
                pltpu.VMEM((TAIL,), jnp.float32)]
  scratch += [pltpu.SemaphoreType.DMA for _ in range(5)]

  @functools.partial(
      pl.kernel, mesh=_mesh,
      out_type=jax.ShapeDtypeStruct((_N,), jnp.float32),
      scratch_types=scratch,
  )
  def k(x_hbm, y_hbm, tab_hbm, out_hbm, *refs):
    xb = refs[0:2]
    yb = refs[2:4]
    ib = refs[4:6]
    gb = refs[6:8]
    pos = 8
    if TAIL:
      xbt, ybt, ibt, gbt = refs[8:12]
      pos = 12
    semL = refs[pos:pos + 2]
    semG = refs[pos + 2]
    semS = refs[pos + 3:pos + 5]

    wid = lax.axis_index("s") * _NC + lax.axis_index("c")

    def cbase(kk):
      return OFF + (wid + kk * _NW) * C

    def guard(kk):
      return wid + kk * _NW < NCH

    @pl.when(guard(0))
    def _():
      pltpu.async_copy(x_hbm.at[pl.ds(cbase(0), C)], xb[0], semL[0])
      pltpu.async_copy(y_hbm.at[pl.ds(cbase(0), C)], yb[0], semL[0])

    for kk in range(K):
      b = kk % 2
      nb = (kk + 1) % 2

      @pl.when(guard(kk))
      def _(kk=kk, b=b):
        pltpu.make_async_copy(x_hbm.at[pl.ds(cbase(kk), C)], xb[b],
                              semL[b]).wait()
        pltpu.make_async_copy(y_hbm.at[pl.ds(cbase(kk), C)], yb[b],
                              semL[b]).wait()

      if kk + 1 < K:
        @pl.when(guard(kk + 1))
        def _(kk=kk, nb=nb):
          pltpu.async_copy(x_hbm.at[pl.ds(cbase(kk + 1), C)], xb[nb],
                           semL[nb])
          pltpu.async_copy(y_hbm.at[pl.ds(cbase(kk + 1), C)], yb[nb],
                           semL[nb])

      @pl.when(guard(kk))
      def _(kk=kk, b=b):
        def step(j, c2):
          sl = pl.ds(j * _L, _L)
          ib[b][sl] = _quantize(xb[b][sl], yb[b][sl])
          return c2

        lax.fori_loop(0, C // _L, step, 0)

      if kk >= 2:
        @pl.when(guard(kk - 2))
        def _(kk=kk, b=b):
          pltpu.make_async_copy(gb[b], out_hbm.at[pl.ds(cbase(kk - 2), C)],
                                semS[b]).wait()

      @pl.when(guard(kk))
      def _(kk=kk, b=b):
        pltpu.async_copy(tab_hbm.at[ib[b]], gb[b], semG).wait()
        pltpu.async_copy(gb[b], out_hbm.at[pl.ds(cbase(kk), C)], semS[b])

    for kk in (K - 2, K - 1):
      b = kk % 2

      @pl.when(guard(kk))
      def _(kk=kk, b=b):
        pltpu.make_async_copy(gb[b], out_hbm.at[pl.ds(cbase(kk), C)],
                              semS[b]).wait()

    if TAIL:
      @pl.when(wid == NCH % _NW)
      def _():
        tbase = OFF + NCH * C
        pltpu.sync_copy(x_hbm.at[pl.ds(tbase, TAIL)], xbt)
        pltpu.sync_copy(y_hbm.at[pl.ds(tbase, TAIL)], ybt)

        def step(j, c2):
          sl = pl.ds(j * _L, _L)
          ibt[sl] = _quantize(xbt[sl], ybt[sl])
          return c2

        lax.fori_loop(0, TAIL // _L, step, 0)
        pltpu.async_copy(tab_hbm.at[ibt], gbt, semG).wait()
        pltpu.sync_copy(gbt, out_hbm.at[pl.ds(tbase, TAIL)])

  return k


def _make_sc_hits(C, OFF, NCH, TAIL):
  K = (NCH + _NW - 1) // _NW
  scratch = (
      [pltpu.VMEM((C,), jnp.float32) for _ in range(2)]
      + [pltpu.VMEM((C,), jnp.float32) for _ in range(2)]
      + [pltpu.VMEM((C,), jnp.float32) for _ in range(2)]
      + [pltpu.VMEM((C,), jnp.float32) for _ in range(2)]
      + [pltpu.VMEM((C,), jnp.int32) for _ in range(2)]
      + [pltpu.VMEM((C,), jnp.float32) for _ in range(2)]
      + [pltpu.VMEM((C,), jnp.float32) for _ in range(2)]
      + [pltpu.VMEM((C,), jnp.float32) for _ in range(2)]
  )
  if TAIL:
    scratch += ([pltpu.VMEM((TAIL,), jnp.float32) for _ in range(4)]
                + [pltpu.VMEM((TAIL,), jnp.int32)]
                + [pltpu.VMEM((TAIL,), jnp.float32) for _ in range(3)])
  scratch += [pltpu.SemaphoreType.DMA for _ in range(5)]

  @functools.partial(
      pl.kernel, mesh=_mesh,
      out_type=(jax.ShapeDtypeStruct((_N,), jnp.float32),
                jax.ShapeDtypeStruct((_N,), jnp.float32)),
      scratch_types=scratch,
  )
  def k(xp_hbm, yp_hbm, nx_hbm, ny_hbm, tab_hbm, hx_hbm, hy_hbm, *refs):
    xpb = refs[0:2]
    ypb = refs[2:4]
    nxb = refs[4:6]
    nyb = refs[6:8]
    f2b = refs[8:10]
    resb = refs[10:12]
    hxb = refs[12:14]
    hyb = refs[14:16]
    pos = 16
    if TAIL:
      xpt, ypt, nxt, nyt, f2t, rest, hxt, hyt = refs[16:24]
      pos = 24
    semL = refs[pos:pos + 2]
    semG = refs[pos + 2]
    semS = refs[pos + 3:pos + 5]

    wid = lax.axis_index("s") * _NC + lax.axis_index("c")
    ins = (xp_hbm, yp_hbm, nx_hbm, ny_hbm)

    def bufs(b):
      return (xpb[b], ypb[b], nxb[b], nyb[b])

    def cbase(kk):
      return OFF + (wid + kk * _NW) * C

    def guard(kk):
      return wid + kk * _NW < NCH

    def issue_loads(kk, b):
      sl = pl.ds(cbase(kk), C)
      for src, dst in zip(ins, bufs(b)):
        pltpu.async_copy(src.at[sl], dst, semL[b])

    def wait_loads(kk, b):
      sl = pl.ds(cbase(kk), C)
      for src, dst in zip(ins, bufs(b)):
        pltpu.make_async_copy(src.at[sl], dst, semL[b]).wait()

    @pl.when(guard(0))
    def _():
      issue_loads(0, 0)

    for kk in range(K):
      b = kk % 2
      nb = (kk + 1) % 2

      @pl.when(guard(kk))
      def _(kk=kk, b=b):
        wait_loads(kk, b)

      if kk + 1 < K:
        @pl.when(guard(kk + 1))
        def _(kk=kk, nb=nb):
          issue_loads(kk + 1, nb)

      if kk >= 2:
        @pl.when(guard(kk - 2))
        def _(kk=kk, b=b):
          pltpu.make_async_copy(hxb[b], hx_hbm.at[pl.ds(cbase(kk - 2), C)],
                                semS[b]).wait()
          pltpu.make_async_copy(hyb[b], hy_hbm.at[pl.ds(cbase(kk - 2), C)],
                                semS[b]).wait()

      @pl.when(guard(kk))
      def _(kk=kk, b=b):
        def qstep(j, c2):
          sl = pl.ds(j * _L, _L)
          xv = xpb[b][sl]
          yv = ypb[b][sl]
          inb = (xv >= 0.0) & (xv < _LW) & (yv >= 0.0) & (yv < _LW)
          f2b[b][sl] = jnp.where(inb, _quantize(xv, yv), _SENT)
          return c2

        lax.fori_loop(0, C // _L, qstep, 0)
        pltpu.async_copy(tab_hbm.at[f2b[b]], resb[b], semG).wait()

        def step(j, c2):
          sl = pl.ds(j * _L, _L)
          d = jnp.abs(resb[b][sl]) + 1e-17
          hxb[b][sl] = xpb[b][sl] + nxb[b][sl] / d
          hyb[b][sl] = ypb[b][sl] + nyb[b][sl] / d
          return c2

        lax.fori_loop(0, C // _L, step, 0)
        pltpu.async_copy(hxb[b], hx_hbm.at[pl.ds(cbase(kk), C)], semS[b])
        pltpu.async_copy(hyb[b], hy_hbm.at[pl.ds(cbase(kk), C)], semS[b])

    for kk in (K - 2, K - 1):
      b = kk % 2

      @pl.when(guard(kk))
      def _(kk=kk, b=b):
        pltpu.make_async_copy(hxb[b], hx_hbm.at[pl.ds(cbase(kk), C)],
                              semS[b]).wait()
        pltpu.make_async_copy(hyb[b], hy_hbm.at[pl.ds(cbase(kk), C)],
                              semS[b]).wait()

    if TAIL:
      @pl.when(wid == NCH % _NW)
      def _():
        tbase = OFF + NCH * C
        sl = pl.ds(tbase, TAIL)
        pltpu.sync_copy(xp_hbm.at[sl], xpt)
        pltpu.sync_copy(yp_hbm.at[sl], ypt)
        pltpu.sync_copy(nx_hbm.at[sl], nxt)
        pltpu.sync_copy(ny_hbm.at[sl], nyt)

        def qstep(j, c2):
          s = pl.ds(j * _L, _L)
          xv = xpt[s]
          yv = ypt[s]
          inb = (xv >= 0.0) & (xv < _LW) & (yv >= 0.0) & (yv < _LW)
          f2t[s] = jnp.where(inb, _quantize(xv, yv), _SENT)
          return c2

        lax.fori_loop(0, TAIL // _L, qstep, 0)
        pltpu.async_copy(tab_hbm.at[f2t], rest, semG).wait()

        def step(j, c2):
          s = pl.ds(j * _L, _L)
          d = jnp.abs(rest[s]) + 1e-17
          hxt[s] = xpt[s] + nxt[s] / d
          hyt[s] = ypt[s] + nyt[s] / d
          return c2

        lax.fori_loop(0, TAIL // _L, step, 0)
        pltpu.sync_copy(hxt, hx_hbm.at[sl])
        pltpu.sync_copy(hyt, hy_hbm.at[sl])

  return k


def _tc_math_body(x_ref, y_ref, th_ref, tx_ref, ty_ref, p_ref, z1_ref, z2_ref,
                  u_ref, rl_ref, xp_ref, yp_ref):
  x = x_ref[...]
  y = y_ref[...]
  theta = th_ref[...]
  theta_x = tx_ref[...]
  theta_y = ty_ref[...]
  p = p_ref[...]
  z1 = z1_ref[...]
  z2 = z2_ref[...]
  u = u_ref[...]
  rl = rl_ref[...]

  mask = (x >= 0.0) & (x < _LW) & (y >= 0.0) & (y < _LW)
  x0 = _DZ / (rl * jnp.cos(theta))
  theta0 = _A / p * jnp.sqrt(x0)
  phi = u * 2.0 * math.pi
  dh = _DZ * jnp.sin(theta0) * (z1 / math.sqrt(12.0) + z2 / 2.0)
  dx = math.sqrt(2.0) * dh * jnp.cos(phi) * jnp.cos(theta_x)
  dy = math.sqrt(2.0) * dh * jnp.sin(phi) * jnp.cos(theta_y)
  xn = jnp.where(mask, x + dx, x)
  yn = jnp.where(mask, y + dy, y)
  xn = xn + _DZ * jnp.tan(theta_x)
  yn = yn + _DZ * jnp.tan(theta_y)

  xp_ref[...] = xn
  yp_ref[...] = yn


def _make_tc_math(grid, block_off):
  spec = pl.BlockSpec((_TB,), lambda i, o=block_off: (i + o,))
  return pl.pallas_call(
      _tc_math_body,
      grid=(grid,),
      in_specs=[spec] * 10,
      out_specs=[spec] * 2,
      out_shape=[
          jax.ShapeDtypeStruct((_N,), jnp.float32),
          jax.ShapeDtypeStruct((_N,), jnp.float32),
      ],
  )


_SLICES = (
    (0, (8192, 128, 0), (4096, 256, 0), (8, 0)),
    (_SLICE_A, (8192, 116, 1152), (4096, 232, 1152), (8, 8)),
)

_sc_rl_k = [_make_sc_rl(c, off, nch, tl)
            for off, (c, nch, tl), _, _ in _SLICES]
_sc_hits_k = [_make_sc_hits(c, off, nch, tl)
              for off, _, (c, nch, tl), _ in _SLICES]
_tc_math_k = [_make_tc_math(g, bo) for _, _, _, (g, bo) in _SLICES]


def kernel(x, y, theta, theta_x, theta_y, p, z1a, z2a, ua, z1b, z2b, ub,
           nx, ny, resolution, efficiency, rad_length):
  tab1 = rad_length.reshape(-1)
  tab2 = jnp.concatenate(
      [resolution.reshape(-1), jnp.zeros((8,), jnp.float32)])

  rls = [k(x, y, tab1) for k in _sc_rl_k]
  xys = [k(x, y, theta, theta_x, theta_y, p, z1a, z2a, ua, rl)
         for k, rl in zip(_tc_math_k, rls)]
  hs = [k(xp, yp, nx, ny, tab2)
        for k, (xp, yp) in zip(_sc_hits_k, xys)]

  pos = lax.broadcasted_iota(jnp.int32, (_N,), 0)
  bounds = [off for off, _, _, _ in _SLICES[1:]]
  hx, hy = hs[-1]
  for bound, (hxs, hys) in zip(reversed(bounds), reversed(hs[:-1])):
    sel = pos < bound
    hx = jnp.where(sel, hxs, hx)
    hy = jnp.where(sel, hys, hy)
  return jnp.stack([hx, hy], axis=1)
